# Initial kernel scaffold; baseline (speedup 1.0000x reference)
import jax
import jax.numpy as jnp
from jax.experimental import pallas as pl


def kernel(x, batch_ids, x_glob):
    raise NotImplementedError("write your pallas kernel here")


# speedup vs baseline: 1.1311x; 1.1311x over previous
import functools

import jax
import jax.numpy as jnp
from jax import lax
from jax.experimental import pallas as pl
from jax.experimental.pallas import tpu as pltpu
from jax.experimental.pallas import tpu_sc as plsc

_N, _B, _D = 524288, 32, 128
_NC, _NS = 2, 16
_NW = _NC * _NS
_ROWS_PER_W = _N // _NW
_CHUNK = 128
_N_CHUNKS = _ROWS_PER_W // _CHUNK


def _sc_broadcast(x_glob, ids):
    mesh = plsc.VectorSubcoreMesh(core_axis_name="c", subcore_axis_name="s")

    @functools.partial(
        pl.kernel,
        mesh=mesh,
        out_type=jax.ShapeDtypeStruct((_N, _D), jnp.float32),
        scratch_types=[
            pltpu.VMEM((_CHUNK,), jnp.int32),
            pltpu.VMEM((_CHUNK, _D), jnp.float32),
            pltpu.SemaphoreType.DMA,
        ],
    )
    def k(table_hbm, idx_hbm, out_hbm, idx_v, rows_v, sem):
        wid = lax.axis_index("s") * _NC + lax.axis_index("c")
        base = wid * _ROWS_PER_W

        def body(i, carry):
            off = base + i * _CHUNK
            pltpu.sync_copy(idx_hbm.at[pl.ds(off, _CHUNK)], idx_v)
            pltpu.async_copy(table_hbm.at[idx_v], rows_v, sem).wait()
            pltpu.sync_copy(rows_v, out_hbm.at[pl.ds(off, _CHUNK)])
            return carry

        lax.fori_loop(0, _N_CHUNKS, body, 0)

    return k(x_glob, ids)


def kernel(x, batch_ids, x_glob):
    del x
    ids = batch_ids.astype(jnp.int32)
    return _sc_broadcast(x_glob, ids)


# speedup vs baseline: 1.2990x; 1.1485x over previous
import functools

import jax
import jax.numpy as jnp
from jax import lax
from jax.experimental import pallas as pl
from jax.experimental.pallas import tpu as pltpu
from jax.experimental.pallas import tpu_sc as plsc

_N, _B, _D = 524288, 32, 128
_NC, _NS = 2, 16
_NW = _NC * _NS
_ROWS_PER_W = _N // _NW
_CHUNK = 128
_SUB = 2
_BIG = _CHUNK * _SUB
_NBIG = _ROWS_PER_W // _BIG
_NBUF = 2
_NGRP = _NBIG // _NBUF


def _sc_broadcast(x_glob, ids2d):
    mesh = plsc.VectorSubcoreMesh(core_axis_name="c", subcore_axis_name="s")

    @functools.partial(
        pl.kernel,
        mesh=mesh,
        out_type=jax.ShapeDtypeStruct((_N, _D), jnp.float32),
        scratch_types=[
            pltpu.VMEM((_ROWS_PER_W // _CHUNK, _CHUNK), jnp.int32),
            pltpu.VMEM((_BIG, _D), jnp.float32),
            pltpu.VMEM((_BIG, _D), jnp.float32),
            pltpu.SemaphoreType.DMA,
            pltpu.SemaphoreType.DMA,
            pltpu.SemaphoreType.DMA,
            pltpu.SemaphoreType.DMA,
        ],
    )
    def k(table_hbm, idx_hbm, out_hbm, idx_v, rows0, rows1, sg0, sg1, ss0, ss1):
        wid = lax.axis_index("s") * _NC + lax.axis_index("c")
        base = wid * _ROWS_PER_W
        rows = (rows0, rows1)
        sg = (sg0, sg1)
        ss = (ss0, ss1)

        pltpu.sync_copy(idx_hbm.at[pl.ds(wid * (_ROWS_PER_W // _CHUNK),
                                         _ROWS_PER_W // _CHUNK)], idx_v)

        def gather_start(b, kk):
            for j in range(_SUB):
                pltpu.async_copy(table_hbm.at[idx_v.at[kk * _SUB + j]],
                                 rows[b].at[pl.ds(j * _CHUNK, _CHUNK)], sg[b])

        def gather_wait(b, kk):
            for j in range(_SUB):
                pltpu.make_async_copy(table_hbm.at[idx_v.at[kk * _SUB + j]],
                                      rows[b].at[pl.ds(j * _CHUNK, _CHUNK)],
                                      sg[b]).wait()

        def scatter_start(b, kk):
            pltpu.async_copy(rows[b], out_hbm.at[pl.ds(base + kk * _BIG, _BIG)],
                             ss[b])

        def scatter_wait(b, kk):
            pltpu.make_async_copy(rows[b],
                                  out_hbm.at[pl.ds(base + kk * _BIG, _BIG)],
                                  ss[b]).wait()

        for b in range(_NBUF):
            gather_start(b, b)

        def body(g, carry):
            for b in range(_NBUF):
                kk = g * _NBUF + b
                gather_wait(b, kk)
                scatter_start(b, kk)
            for b in range(_NBUF):
                kk = g * _NBUF + b
                scatter_wait(b, kk)
                gather_start(b, kk + _NBUF)
            return carry

        lax.fori_loop(0, _NGRP - 1, body, 0)

        for b in range(_NBUF):
            kk = (_NGRP - 1) * _NBUF + b
            gather_wait(b, kk)
            scatter_start(b, kk)
        for b in range(_NBUF):
            kk = (_NGRP - 1) * _NBUF + b
            scatter_wait(b, kk)

    return k(x_glob, ids2d)


def kernel(x, batch_ids, x_glob):
    del x
    ids2d = batch_ids.astype(jnp.int32).reshape(_N // _CHUNK, _CHUNK)
    return _sc_broadcast(x_glob, ids2d)


# speedup vs baseline: 10.1564x; 7.8186x over previous
import functools

import jax
import jax.numpy as jnp
from jax import lax
from jax.experimental import pallas as pl
from jax.experimental.pallas import tpu as pltpu
from jax.experimental.pallas import tpu_sc as plsc

_N, _B, _D = 524288, 32, 128
_NC, _NS = 2, 16
_NW = _NC * _NS
_ROWS_PER_W = _N // _NW
_IDXCHUNK = 128
_BIGR = 256
_NCHUNK = _ROWS_PER_W // _BIGR
_MAXPEND = 4


def _sc_broadcast(x_glob, ids):
    mesh = plsc.VectorSubcoreMesh(core_axis_name="c", subcore_axis_name="s")

    @functools.partial(
        pl.kernel,
        mesh=mesh,
        out_type=jax.ShapeDtypeStruct((_N, _D), jnp.float32),
        scratch_types=[
            pltpu.VMEM((_ROWS_PER_W,), jnp.int32),
            pltpu.VMEM((_BIGR, _D), jnp.float32),
            pltpu.VMEM((_BIGR, _D), jnp.float32),
            pltpu.SemaphoreType.DMA,
            pltpu.SemaphoreType.DMA,
        ],
    )
    def k(table_hbm, idx_hbm, out_hbm, idx_v, run_buf, alt_buf, sg, ss):
        wid = lax.axis_index("s") * _NC + lax.axis_index("c")
        base = wid * _ROWS_PER_W

        pltpu.sync_copy(idx_hbm.at[pl.ds(base, _ROWS_PER_W)], idx_v)

        def store_start(buf, kk):
            pltpu.async_copy(buf, out_hbm.at[pl.ds(base + kk * _BIGR, _BIGR)],
                             ss)

        def store_wait_one():
            pltpu.make_async_copy(run_buf, out_hbm.at[pl.ds(base, _BIGR)],
                                  ss).wait()

        def drain_all(npend):
            lax.fori_loop(0, npend, lambda i, c: (store_wait_one(), c)[1], 0)
            return jnp.int32(0)

        def cap_pending(npend):
            def w(n):
                store_wait_one()
                return n - 1
            return lax.cond(npend >= _MAXPEND, w, lambda n: n, npend)

        def refill(bid):
            idx16 = jnp.full((16,), bid, dtype=jnp.int32)
            pltpu.async_copy(table_hbm.at[idx16], run_buf.at[pl.ds(0, 16)],
                             sg).wait()
            row = [run_buf[0, pl.ds(16 * j, 16)] for j in range(_D // 16)]

            def rep(r, c):
                for j in range(_D // 16):
                    run_buf[r, pl.ds(16 * j, 16)] = row[j]
                return c

            lax.fori_loop(16, _BIGR, rep, 0)

        def chunk(kk, carry):
            cur_id, npend = carry
            off = kk * _BIGR
            mns = idx_v[pl.ds(off, 16)][0]
            mxs = idx_v[pl.ds(off + _BIGR - 16, 16)][15]

            def uniform_case(cur_id, npend):
                def same(cur_id, npend):
                    npend = cap_pending(npend)
                    store_start(run_buf, kk)
                    return cur_id, npend + 1

                def diff(cur_id, npend):
                    npend = drain_all(npend)
                    refill(mns)
                    store_start(run_buf, kk)
                    return mns, jnp.int32(1)

                return lax.cond(mns == cur_id, same, diff, cur_id, npend)

            def boundary_case(cur_id, npend):
                npend = drain_all(npend)
                for j in range(_BIGR // _IDXCHUNK):
                    pltpu.async_copy(
                        table_hbm.at[idx_v.at[pl.ds(off + j * _IDXCHUNK,
                                                    _IDXCHUNK)]],
                        alt_buf.at[pl.ds(j * _IDXCHUNK, _IDXCHUNK)], sg)
                for j in range(_BIGR // _IDXCHUNK):
                    pltpu.make_async_copy(
                        table_hbm.at[idx_v.at[pl.ds(off + j * _IDXCHUNK,
                                                    _IDXCHUNK)]],
                        alt_buf.at[pl.ds(j * _IDXCHUNK, _IDXCHUNK)],
                        sg).wait()
                store_start(alt_buf, kk)
                return cur_id, npend + 1

            return lax.cond(mns == mxs, uniform_case, boundary_case,
                            cur_id, npend)

        cur_id, npend = lax.fori_loop(0, _NCHUNK, chunk,
                                      (jnp.int32(-1), jnp.int32(0)))
        drain_all(npend)

    return k(x_glob, ids)


def kernel(x, batch_ids, x_glob):
    del x
    ids = batch_ids.astype(jnp.int32)
    return _sc_broadcast(x_glob, ids)


# speedup vs baseline: 17.4140x; 1.7146x over previous
import functools

import jax
import jax.numpy as jnp
from jax import lax
from jax.experimental import pallas as pl
from jax.experimental.pallas import tpu as pltpu
from jax.experimental.pallas import tpu_sc as plsc

_N, _B, _D = 524288, 32, 128
_NC, _NS = 2, 16
_NW = _NC * _NS
_RPW = _N // _NW
_NBLK = _RPW // 16
_BIG = 256
_MAXPEND = 8


def _sc_broadcast(x_glob, ids):
    mesh = plsc.VectorSubcoreMesh(core_axis_name="c", subcore_axis_name="s")

    @functools.partial(
        pl.kernel,
        mesh=mesh,
        out_type=jax.ShapeDtypeStruct((_N * _D,), jnp.float32),
        scratch_types=[
            pltpu.VMEM((_RPW + 16,), jnp.int32),
            pltpu.VMEM((_BIG * _D,), jnp.float32),
            pltpu.VMEM((16, _D), jnp.float32),
            pltpu.SemaphoreType.DMA,
            pltpu.SemaphoreType.DMA,
            pltpu.SemaphoreType.DMA,
        ],
    )
    def k(table_hbm, idx_hbm, out_hbm, idx_v, run_buf, gbuf, sg, ss, st):
        wid = lax.axis_index("s") * _NC + lax.axis_index("c")
        base = wid * _RPW

        pltpu.sync_copy(idx_hbm.at[pl.ds(base, _RPW)],
                        idx_v.at[pl.ds(0, _RPW)])
        idx_v[pl.ds(_RPW, 16)] = jnp.full((16,), _B, dtype=jnp.int32)

        def store_wait_one():
            pltpu.make_async_copy(run_buf,
                                  out_hbm.at[pl.ds(base * _D, _BIG * _D)],
                                  ss).wait()

        def drain_all(npend):
            lax.fori_loop(0, npend, lambda i, c: (store_wait_one(), c)[1], 0)
            return jnp.int32(0)

        def cap_pending(npend):
            def w(n):
                store_wait_one()
                return n - 1
            return lax.cond(npend >= _MAXPEND, w, lambda n: n, npend)

        def big_store(pos, npend):
            npend = cap_pending(npend)
            pltpu.async_copy(run_buf,
                             out_hbm.at[pl.ds((base + pos) * _D, _BIG * _D)],
                             ss)
            return npend + 1

        def small_store16(pos):
            pltpu.async_copy(run_buf.at[pl.ds(0, 16 * _D)],
                             out_hbm.at[pl.ds((base + pos) * _D, 16 * _D)],
                             st).wait()

        def small_store1(pos):
            pltpu.async_copy(run_buf.at[pl.ds(0, _D)],
                             out_hbm.at[pl.ds((base + pos) * _D, _D)],
                             st).wait()

        def refill(bid):
            idx16 = jnp.full((16,), bid, dtype=jnp.int32)
            pltpu.async_copy(table_hbm.at[idx16], gbuf, sg).wait()
            row = [gbuf[0, pl.ds(16 * j, 16)] for j in range(_D // 16)]

            def rep(r, c):
                for j in range(_D // 16):
                    run_buf[pl.ds(r * _D + 16 * j, 16)] = row[j]
                return c

            lax.fori_loop(0, _BIG, rep, 0)

        def find_run_end(pos, vid):
            def body_f(i, c):
                lo, hi = c
                mid = (lo + hi) // 2
                gt = idx_v[pl.ds(16 * mid, 16)][0] > vid
                return jnp.where(gt, lo, mid), jnp.where(gt, mid, hi)

            lo, hi = lax.fori_loop(
                0, _NBLK.bit_length(), body_f,
                (pos // 16, jnp.int32(_NBLK)))
            blk = idx_v[pl.ds(16 * lo, 16)]
            cnt = jnp.int32(0)
            for j in range(16):
                cnt = cnt + jnp.where(blk[j] <= vid, 1, 0).astype(jnp.int32)
            return 16 * lo + cnt

        def run_body(pos, cur, npend):
            vid = idx_v[pl.ds(pos, 16)][0]
            run_end = find_run_end(pos, vid)
            ln = run_end - pos

            def do_refill(npend):
                npend = drain_all(npend)
                refill(vid)
                return npend

            npend = lax.cond(vid != cur, do_refill, lambda n: n, npend)

            def big_path(npend):
                nbig = ln // _BIG
                npend = lax.fori_loop(
                    0, nbig, lambda i, n: big_store(pos + i * _BIG, n), npend)
                return lax.cond(ln % _BIG > 0,
                                lambda n: big_store(run_end - _BIG, n),
                                lambda n: n, npend)

            def small_path(npend):
                n16 = ln // 16
                lax.fori_loop(0, n16,
                              lambda i, c: (small_store16(pos + 16 * i), c)[1],
                              0)

                def tail(c):
                    def overlap(c):
                        small_store16(run_end - 16)
                        return c

                    def ones(c):
                        return lax.fori_loop(
                            0, ln % 16,
                            lambda i, cc: (small_store1(pos + 16 * n16 + i),
                                           cc)[1], c)

                    return lax.cond(ln >= 16, overlap, ones, c)

                lax.cond(ln % 16 > 0, tail, lambda c: c, 0)
                return npend

            npend = lax.cond(ln >= _BIG, big_path, small_path, npend)
            return run_end, vid, npend

        def outer(i, carry):
            pos, cur, npend = carry
            return lax.cond(pos < _RPW,
                            run_body,
                            lambda p, c, n: (p, c, n),
                            pos, cur, npend)

        pos, cur, npend = lax.fori_loop(
            0, _B + 1, outer, (jnp.int32(0), jnp.int32(-1), jnp.int32(0)))
        drain_all(npend)

    return k(x_glob, ids)


def kernel(x, batch_ids, x_glob):
    del x
    ids = batch_ids.astype(jnp.int32)
    return _sc_broadcast(x_glob, ids).reshape(_N, _D)


# speedup vs baseline: 17.8396x; 1.0244x over previous
import functools

import jax
import jax.numpy as jnp
from jax import lax
from jax.experimental import pallas as pl
from jax.experimental.pallas import tpu as pltpu
from jax.experimental.pallas import tpu_sc as plsc

_N, _B, _D = 524288, 32, 128
_NC, _NS = 2, 16
_NW = _NC * _NS
_RPW = _N // _NW
_NBLK = _RPW // 16
_BIG = 256
_MAXPEND = 8


def _sc_broadcast(x_glob, ids):
    mesh = plsc.VectorSubcoreMesh(core_axis_name="c", subcore_axis_name="s")

    @functools.partial(
        pl.kernel,
        mesh=mesh,
        out_type=jax.ShapeDtypeStruct((_N * _D,), jnp.float32),
        scratch_types=[
            pltpu.VMEM((_RPW + 16,), jnp.int32),
            pltpu.VMEM((_BIG * _D,), jnp.float32),
            pltpu.VMEM((_BIG * _D,), jnp.float32),
            pltpu.VMEM((16, _D), jnp.float32),
            pltpu.VMEM((16, _D), jnp.float32),
            pltpu.SemaphoreType.DMA,
            pltpu.SemaphoreType.DMA,
            pltpu.SemaphoreType.DMA,
            pltpu.SemaphoreType.DMA,
            pltpu.SemaphoreType.DMA,
        ],
    )
    def k(table_hbm, idx_hbm, out_hbm, idx_v, rbuf0, rbuf1, gbuf0, gbuf1,
          sg0, sg1, ss0, ss1, st):
        wid = lax.axis_index("s") * _NC + lax.axis_index("c")
        base = wid * _RPW
        rbuf = (rbuf0, rbuf1)
        gbuf = (gbuf0, gbuf1)
        sg = (sg0, sg1)
        ss = (ss0, ss1)

        pltpu.sync_copy(idx_hbm.at[pl.ds(base, _RPW)],
                        idx_v.at[pl.ds(0, _RPW)])
        idx_v[pl.ds(_RPW, 16)] = jnp.full((16,), _B, dtype=jnp.int32)

        def gather_start(b, bid):
            idx16 = jnp.full((16,), bid, dtype=jnp.int32)
            pltpu.async_copy(table_hbm.at[idx16], gbuf[b], sg[b])

        def gather_wait(b):
            idx16 = jnp.zeros((16,), dtype=jnp.int32)
            pltpu.make_async_copy(table_hbm.at[idx16], gbuf[b], sg[b]).wait()

        def store_wait_one(b):
            pltpu.make_async_copy(rbuf[b],
                                  out_hbm.at[pl.ds(base * _D, _BIG * _D)],
                                  ss[b]).wait()

        def drain_buf(b, npend):
            lax.fori_loop(0, npend, lambda i, c: (store_wait_one(b), c)[1], 0)
            return jnp.int32(0)

        def cap_pending(b, npend):
            def w(n):
                store_wait_one(b)
                return n - 1
            return lax.cond(npend >= _MAXPEND, w, lambda n: n, npend)

        def big_store(b, pos, npend):
            npend = cap_pending(b, npend)
            pltpu.async_copy(rbuf[b],
                             out_hbm.at[pl.ds((base + pos) * _D, _BIG * _D)],
                             ss[b])
            return npend + 1

        def small_store16(b, pos):
            pltpu.async_copy(rbuf[b].at[pl.ds(0, 16 * _D)],
                             out_hbm.at[pl.ds((base + pos) * _D, 16 * _D)],
                             st).wait()

        def small_store1(b, pos):
            pltpu.async_copy(rbuf[b].at[pl.ds(0, _D)],
                             out_hbm.at[pl.ds((base + pos) * _D, _D)],
                             st).wait()

        def replicate(b):
            row = [gbuf[b][0, pl.ds(16 * j, 16)] for j in range(_D // 16)]

            def rep(r, c):
                for j in range(_D // 16):
                    rbuf[b][pl.ds(r * _D + 16 * j, 16)] = row[j]
                return c

            lax.fori_loop(0, _BIG, rep, 0)

        def find_run_end(pos, vid):
            def body_f(i, c):
                lo, hi = c
                mid = (lo + hi) // 2
                gt = idx_v[pl.ds(16 * mid, 16)][0] > vid
                return jnp.where(gt, lo, mid), jnp.where(gt, mid, hi)

            lo, hi = lax.fori_loop(
                0, _NBLK.bit_length(), body_f,
                (pos // 16, jnp.int32(_NBLK)))
            blk = idx_v[pl.ds(16 * lo, 16)]
            cnt = jnp.int32(0)
            for j in range(16):
                cnt = cnt + jnp.where(blk[j] <= vid, 1, 0).astype(jnp.int32)
            return 16 * lo + cnt

        def process(b, pos, vid, run_end, npend):
            npend = drain_buf(b, npend)
            gather_wait(b)
            replicate(b)

            def prefetch(_):
                nvid = idx_v[pl.ds(run_end, 16)][0]
                gather_start(1 - b, nvid)
                return 0

            lax.cond(run_end < _RPW, prefetch, lambda _: 0, 0)

            ln = run_end - pos

            def big_path(npend):
                nbig = ln // _BIG
                npend = lax.fori_loop(
                    0, nbig, lambda i, n: big_store(b, pos + i * _BIG, n),
                    npend)
                return lax.cond(ln % _BIG > 0,
                                lambda n: big_store(b, run_end - _BIG, n),
                                lambda n: n, npend)

            def small_path(npend):
                n16 = ln // 16
                lax.fori_loop(
                    0, n16,
                    lambda i, c: (small_store16(b, pos + 16 * i), c)[1], 0)

                def tail(c):
                    def overlap(c):
                        small_store16(b, run_end - 16)
                        return c

                    def ones(c):
                        return lax.fori_loop(
                            0, ln % 16,
                            lambda i, cc: (small_store1(b, pos + 16 * n16 + i),
                                           cc)[1], c)

                    return lax.cond(ln >= 16, overlap, ones, c)

                lax.cond(ln % 16 > 0, tail, lambda c: c, 0)
                return npend

            return lax.cond(ln >= _BIG, big_path, small_path, npend)

        def run_body(pos, cb, npend0, npend1):
            vid = idx_v[pl.ds(pos, 16)][0]
            run_end = find_run_end(pos, vid)

            def use0(npend0, npend1):
                return process(0, pos, vid, run_end, npend0), npend1

            def use1(npend0, npend1):
                return npend0, process(1, pos, vid, run_end, npend1)

            npend0, npend1 = lax.cond(cb == 0, use0, use1, npend0, npend1)
            return run_end, 1 - cb, npend0, npend1

        gather_start(0, idx_v[pl.ds(0, 16)][0])

        def outer(i, carry):
            pos, cb, npend0, npend1 = carry
            return lax.cond(pos < _RPW,
                            run_body,
                            lambda p, c, n0, n1: (p, c, n0, n1),
                            pos, cb, npend0, npend1)

        pos, cb, npend0, npend1 = lax.fori_loop(
            0, _B, outer,
            (jnp.int32(0), jnp.int32(0), jnp.int32(0), jnp.int32(0)))
        drain_buf(0, npend0)
        drain_buf(1, npend1)

    return k(x_glob, ids)


def kernel(x, batch_ids, x_glob):
    del x
    ids = batch_ids.astype(jnp.int32)
    return _sc_broadcast(x_glob, ids).reshape(_N, _D)


# speedup vs baseline: 17.8484x; 1.0005x over previous
import functools

import jax
import jax.numpy as jnp
from jax import lax
from jax.experimental import pallas as pl
from jax.experimental.pallas import tpu as pltpu
from jax.experimental.pallas import tpu_sc as plsc

_N, _B, _D = 524288, 32, 128
_NC, _NS = 2, 16
_NW = _NC * _NS
_RPW = _N // _NW
_NBLK = _RPW // 16
_BIG = 256
_MAXPEND = 8


def _sc_broadcast(x_glob, ids):
    mesh = plsc.VectorSubcoreMesh(core_axis_name="c", subcore_axis_name="s")

    @functools.partial(
        pl.kernel,
        mesh=mesh,
        out_type=jax.ShapeDtypeStruct((_N * _D,), jnp.float32),
        scratch_types=[
            pltpu.VMEM((_RPW + 16,), jnp.int32),
            pltpu.VMEM((_BIG * _D,), jnp.float32),
            pltpu.VMEM((_BIG * _D,), jnp.float32),
            pltpu.VMEM((16, _D), jnp.float32),
            pltpu.VMEM((16, _D), jnp.float32),
            pltpu.SemaphoreType.DMA,
            pltpu.SemaphoreType.DMA,
            pltpu.SemaphoreType.DMA,
            pltpu.SemaphoreType.DMA,
            pltpu.SemaphoreType.DMA,
        ],
    )
    def k(table_hbm, idx_hbm, out_hbm, idx_v, rbuf0, rbuf1, gbuf0, gbuf1,
          sg0, sg1, ss0, ss1, st):
        wid = lax.axis_index("s") * _NC + lax.axis_index("c")
        base = wid * _RPW
        rbuf = (rbuf0, rbuf1)
        gbuf = (gbuf0, gbuf1)
        sg = (sg0, sg1)
        ss = (ss0, ss1)

        pltpu.sync_copy(idx_hbm.at[pl.ds(base, 16)], idx_v.at[pl.ds(0, 16)])

        def gather_start(b, bid):
            idx16 = jnp.full((16,), bid, dtype=jnp.int32)
            pltpu.async_copy(table_hbm.at[idx16], gbuf[b], sg[b])

        def gather_wait(b):
            idx16 = jnp.zeros((16,), dtype=jnp.int32)
            pltpu.make_async_copy(table_hbm.at[idx16], gbuf[b], sg[b]).wait()

        def store_wait_one(b):
            pltpu.make_async_copy(rbuf[b],
                                  out_hbm.at[pl.ds(base * _D, _BIG * _D)],
                                  ss[b]).wait()

        def drain_buf(b, npend):
            lax.fori_loop(0, npend, lambda i, c: (store_wait_one(b), c)[1], 0)
            return jnp.int32(0)

        def cap_pending(b, npend):
            def w(n):
                store_wait_one(b)
                return n - 1
            return lax.cond(npend >= _MAXPEND, w, lambda n: n, npend)

        def big_store(b, pos, npend):
            npend = cap_pending(b, npend)
            pltpu.async_copy(rbuf[b],
                             out_hbm.at[pl.ds((base + pos) * _D, _BIG * _D)],
                             ss[b])
            return npend + 1

        def small_store16(b, pos):
            pltpu.async_copy(rbuf[b].at[pl.ds(0, 16 * _D)],
                             out_hbm.at[pl.ds((base + pos) * _D, 16 * _D)],
                             st).wait()

        def small_store1(b, pos):
            pltpu.async_copy(rbuf[b].at[pl.ds(0, _D)],
                             out_hbm.at[pl.ds((base + pos) * _D, _D)],
                             st).wait()

        def replicate(b):
            row = [gbuf[b][0, pl.ds(16 * j, 16)] for j in range(_D // 16)]

            def rep(r, c):
                for u in range(2):
                    for j in range(_D // 16):
                        rbuf[b][pl.ds((2 * r + u) * _D + 16 * j, 16)] = row[j]
                return c

            lax.fori_loop(0, _BIG // 2, rep, 0)

        def find_run_end(pos, vid):
            def body_f(i, c):
                lo, hi = c
                mid = (lo + hi) // 2
                gt = idx_v[pl.ds(16 * mid, 16)][0] > vid
                return jnp.where(gt, lo, mid), jnp.where(gt, mid, hi)

            lo, hi = lax.fori_loop(
                0, _NBLK.bit_length(), body_f,
                (pos // 16, jnp.int32(_NBLK)))
            blk = idx_v[pl.ds(16 * lo, 16)]
            cnt = jnp.int32(0)
            for j in range(16):
                cnt = cnt + jnp.where(blk[j] <= vid, 1, 0).astype(jnp.int32)
            return 16 * lo + cnt

        def process(b, pos, vid, run_end, npend):
            npend = drain_buf(b, npend)
            gather_wait(b)
            replicate(b)

            def prefetch(_):
                nvid = idx_v[pl.ds(run_end, 16)][0]
                gather_start(1 - b, nvid)
                return 0

            lax.cond(run_end < _RPW, prefetch, lambda _: 0, 0)

            ln = run_end - pos

            def big_path(npend):
                nbig = ln // _BIG
                npend = lax.fori_loop(
                    0, nbig, lambda i, n: big_store(b, pos + i * _BIG, n),
                    npend)
                return lax.cond(ln % _BIG > 0,
                                lambda n: big_store(b, run_end - _BIG, n),
                                lambda n: n, npend)

            def small_path(npend):
                n16 = ln // 16
                lax.fori_loop(
                    0, n16,
                    lambda i, c: (small_store16(b, pos + 16 * i), c)[1], 0)

                def tail(c):
                    def overlap(c):
                        small_store16(b, run_end - 16)
                        return c

                    def ones(c):
                        return lax.fori_loop(
                            0, ln % 16,
                            lambda i, cc: (small_store1(b, pos + 16 * n16 + i),
                                           cc)[1], c)

                    return lax.cond(ln >= 16, overlap, ones, c)

                lax.cond(ln % 16 > 0, tail, lambda c: c, 0)
                return npend

            return lax.cond(ln >= _BIG, big_path, small_path, npend)

        def run_body(pos, cb, npend0, npend1):
            vid = idx_v[pl.ds(pos, 16)][0]
            run_end = find_run_end(pos, vid)

            def use0(npend0, npend1):
                return process(0, pos, vid, run_end, npend0), npend1

            def use1(npend0, npend1):
                return npend0, process(1, pos, vid, run_end, npend1)

            npend0, npend1 = lax.cond(cb == 0, use0, use1, npend0, npend1)
            return run_end, 1 - cb, npend0, npend1

        gather_start(0, idx_v[pl.ds(0, 16)][0])
        pltpu.sync_copy(idx_hbm.at[pl.ds(base + 16, _RPW - 16)],
                        idx_v.at[pl.ds(16, _RPW - 16)])
        idx_v[pl.ds(_RPW, 16)] = jnp.full((16,), _B, dtype=jnp.int32)

        def outer(i, carry):
            pos, cb, npend0, npend1 = carry
            return lax.cond(pos < _RPW,
                            run_body,
                            lambda p, c, n0, n1: (p, c, n0, n1),
                            pos, cb, npend0, npend1)

        pos, cb, npend0, npend1 = lax.fori_loop(
            0, _B, outer,
            (jnp.int32(0), jnp.int32(0), jnp.int32(0), jnp.int32(0)))
        drain_buf(0, npend0)
        drain_buf(1, npend1)

    return k(x_glob, ids)


def kernel(x, batch_ids, x_glob):
    del x
    ids = batch_ids.astype(jnp.int32)
    return _sc_broadcast(x_glob, ids).reshape(_N, _D)


# speedup vs baseline: 17.9646x; 1.0065x over previous
import functools

import jax
import jax.numpy as jnp
from jax import lax
from jax.experimental import pallas as pl
from jax.experimental.pallas import tpu as pltpu
from jax.experimental.pallas import tpu_sc as plsc

_N, _B, _D = 524288, 32, 128
_NC, _NS = 2, 16
_NW = _NC * _NS
_RPW = _N // _NW
_NBLK = _RPW // 16
_BIG = 256
_MAXPEND = 8


def _sc_broadcast(x_glob, ids):
    mesh = plsc.VectorSubcoreMesh(core_axis_name="c", subcore_axis_name="s")

    @functools.partial(
        pl.kernel,
        mesh=mesh,
        out_type=jax.ShapeDtypeStruct((_N * _D,), jnp.float32),
        scratch_types=[
            pltpu.VMEM((_RPW + 16,), jnp.int32),
            pltpu.VMEM((_BIG * _D,), jnp.float32),
            pltpu.VMEM((_BIG * _D,), jnp.float32),
            pltpu.VMEM((16, _D), jnp.float32),
            pltpu.VMEM((16, _D), jnp.float32),
            pltpu.SemaphoreType.DMA,
            pltpu.SemaphoreType.DMA,
            pltpu.SemaphoreType.DMA,
            pltpu.SemaphoreType.DMA,
            pltpu.SemaphoreType.DMA,
        ],
    )
    def k(table_hbm, idx_hbm, out_hbm, idx_v, rbuf0, rbuf1, gbuf0, gbuf1,
          sg0, sg1, ss0, ss1, st):
        wid = lax.axis_index("s") * _NC + lax.axis_index("c")
        base = wid * _RPW
        rbuf = (rbuf0, rbuf1)
        gbuf = (gbuf0, gbuf1)
        sg = (sg0, sg1)
        ss = (ss0, ss1)

        pltpu.sync_copy(idx_hbm.at[pl.ds(base, 16)], idx_v.at[pl.ds(0, 16)])

        def gather_start(b, bid):
            idx16 = jnp.full((16,), bid, dtype=jnp.int32)
            pltpu.async_copy(table_hbm.at[idx16], gbuf[b], sg[b])

        def gather_wait(b):
            idx16 = jnp.zeros((16,), dtype=jnp.int32)
            pltpu.make_async_copy(table_hbm.at[idx16], gbuf[b], sg[b]).wait()

        def store_wait_one(b):
            pltpu.make_async_copy(rbuf[b],
                                  out_hbm.at[pl.ds(base * _D, _BIG * _D)],
                                  ss[b]).wait()

        def drain_buf(b, npend):
            lax.fori_loop(0, npend, lambda i, c: (store_wait_one(b), c)[1], 0)
            return jnp.int32(0)

        def cap_pending(b, npend):
            def w(n):
                store_wait_one(b)
                return n - 1
            return lax.cond(npend >= _MAXPEND, w, lambda n: n, npend)

        def big_store(b, pos, npend):
            npend = cap_pending(b, npend)
            pltpu.async_copy(rbuf[b],
                             out_hbm.at[pl.ds((base + pos) * _D, _BIG * _D)],
                             ss[b])
            return npend + 1

        def small_store16(b, pos):
            pltpu.async_copy(rbuf[b].at[pl.ds(0, 16 * _D)],
                             out_hbm.at[pl.ds((base + pos) * _D, 16 * _D)],
                             st).wait()

        def small_store1(b, pos):
            pltpu.async_copy(rbuf[b].at[pl.ds(0, _D)],
                             out_hbm.at[pl.ds((base + pos) * _D, _D)],
                             st).wait()

        def replicate(b):
            row = [gbuf[b][0, pl.ds(16 * j, 16)] for j in range(_D // 16)]

            def rep(r, c):
                for u in range(2):
                    for j in range(_D // 16):
                        rbuf[b][pl.ds((2 * r + u) * _D + 16 * j, 16)] = row[j]
                return c

            lax.fori_loop(0, _BIG // 2, rep, 0)

        def find_run_end(pos, vid):
            def body_f(i, c):
                lo, hi = c
                mid = (lo + hi) // 2
                gt = idx_v[pl.ds(16 * mid, 16)][0] > vid
                return jnp.where(gt, lo, mid), jnp.where(gt, mid, hi)

            lo, hi = lax.fori_loop(
                0, _NBLK.bit_length(), body_f,
                (pos // 16, jnp.int32(_NBLK)))
            blk = idx_v[pl.ds(16 * lo, 16)]
            cnt = jnp.int32(0)
            for j in range(16):
                cnt = cnt + jnp.where(blk[j] <= vid, 1, 0).astype(jnp.int32)
            return 16 * lo + cnt

        def process(b, pos, vid, run_end, npend):
            npend = drain_buf(b, npend)

            def do_refill(_):
                gather_wait(b)
                replicate(b)
                return 0

            lax.cond(pos > 0, do_refill, lambda _: 0, 0)

            def prefetch(_):
                nvid = idx_v[pl.ds(run_end, 16)][0]
                gather_start(1 - b, nvid)
                return 0

            lax.cond(run_end < _RPW, prefetch, lambda _: 0, 0)

            ln = run_end - pos

            def big_path(npend):
                nbig = ln // _BIG
                npend = lax.fori_loop(
                    0, nbig, lambda i, n: big_store(b, pos + i * _BIG, n),
                    npend)
                return lax.cond(ln % _BIG > 0,
                                lambda n: big_store(b, run_end - _BIG, n),
                                lambda n: n, npend)

            def small_path(npend):
                n16 = ln // 16
                lax.fori_loop(
                    0, n16,
                    lambda i, c: (small_store16(b, pos + 16 * i), c)[1], 0)

                def tail(c):
                    def overlap(c):
                        small_store16(b, run_end - 16)
                        return c

                    def ones(c):
                        return lax.fori_loop(
                            0, ln % 16,
                            lambda i, cc: (small_store1(b, pos + 16 * n16 + i),
                                           cc)[1], c)

                    return lax.cond(ln >= 16, overlap, ones, c)

                lax.cond(ln % 16 > 0, tail, lambda c: c, 0)
                return npend

            return lax.cond(ln >= _BIG, big_path, small_path, npend)

        def run_body(pos, cb, npend0, npend1):
            vid = idx_v[pl.ds(pos, 16)][0]
            run_end = find_run_end(pos, vid)

            def use0(npend0, npend1):
                return process(0, pos, vid, run_end, npend0), npend1

            def use1(npend0, npend1):
                return npend0, process(1, pos, vid, run_end, npend1)

            npend0, npend1 = lax.cond(cb == 0, use0, use1, npend0, npend1)
            return run_end, 1 - cb, npend0, npend1

        gather_start(0, idx_v[pl.ds(0, 16)][0])
        pltpu.async_copy(idx_hbm.at[pl.ds(base + 16, _RPW - 16)],
                         idx_v.at[pl.ds(16, _RPW - 16)], st)
        gather_wait(0)
        replicate(0)
        pltpu.make_async_copy(idx_hbm.at[pl.ds(base + 16, _RPW - 16)],
                              idx_v.at[pl.ds(16, _RPW - 16)], st).wait()
        idx_v[pl.ds(_RPW, 16)] = jnp.full((16,), _B, dtype=jnp.int32)

        def outer(i, carry):
            pos, cb, npend0, npend1 = carry
            return lax.cond(pos < _RPW,
                            run_body,
                            lambda p, c, n0, n1: (p, c, n0, n1),
                            pos, cb, npend0, npend1)

        pos, cb, npend0, npend1 = lax.fori_loop(
            0, _B, outer,
            (jnp.int32(0), jnp.int32(0), jnp.int32(0), jnp.int32(0)))
        drain_buf(0, npend0)
        drain_buf(1, npend1)

    return k(x_glob, ids)


def kernel(x, batch_ids, x_glob):
    del x
    ids = batch_ids.astype(jnp.int32)
    return _sc_broadcast(x_glob, ids).reshape(_N, _D)
